# baseline (speedup 1.0000x reference)
import functools

import jax
import jax.numpy as jnp
from jax import lax
from jax.experimental import pallas as pl
from jax.experimental.pallas import tpu as pltpu
from jax.experimental.pallas import tpu_sc as plsc

N_NODES = 10000
D_FEAT = 128
N_EDGES = 320000

NC = 2
NS = 16
NW = NC * NS
EPW = N_EDGES // NW
B_EDGE = 80
EPW_PAD = 10240
NB = EPW_PAD // B_EDGE
ZERO_ROW = N_NODES
ROWS_PER_TILE = 624
ROWS_TAIL = N_NODES - NS * ROWS_PER_TILE


def _segment_sum_sc(x, src3, dst3, zeros):
    mesh = plsc.VectorSubcoreMesh(core_axis_name="c", subcore_axis_name="s")

    @functools.partial(
        pl.kernel,
        mesh=mesh,
        out_type=jax.ShapeDtypeStruct((NC, N_NODES, D_FEAT), jnp.float32),
        scratch_types=[
            pltpu.VMEM((NB, B_EDGE), jnp.int32),
            pltpu.VMEM((B_EDGE, D_FEAT), jnp.float32),
            pltpu.VMEM((B_EDGE, D_FEAT), jnp.float32),
            pltpu.VMEM_SHARED((N_NODES, D_FEAT), jnp.float32),
            pltpu.SemaphoreType.DMA,
            pltpu.SemaphoreType.DMA,
        ],
    )
    def k(x_hbm, src_hbm, dst_hbm, zero_hbm, out_hbm,
          src_v, rows0, rows1, acc, gsem0, gsem1):
        cid = lax.axis_index("c")
        sid = lax.axis_index("s")
        wid = sid * NC + cid
        row0 = sid * ROWS_PER_TILE

        pltpu.sync_copy(zero_hbm.at[pl.ds(0, ROWS_PER_TILE)],
                        acc.at[pl.ds(row0, ROWS_PER_TILE)])

        @pl.when(sid == NS - 1)
        def _():
            pltpu.sync_copy(
                zero_hbm.at[pl.ds(0, ROWS_TAIL)],
                acc.at[pl.ds(NS * ROWS_PER_TILE, ROWS_TAIL)])

        pltpu.sync_copy(src_hbm.at[wid], src_v)
        plsc.subcore_barrier()

        def body(j, carry):
            b = 2 * j
            hg0 = pltpu.async_copy(x_hbm.at[src_v.at[b]], rows0, gsem0)
            hg1 = pltpu.async_copy(x_hbm.at[src_v.at[b + 1]], rows1, gsem1)
            hg0.wait()
            hg1.wait()
            return carry

        lax.fori_loop(0, NB // 2, body, 0)
        plsc.subcore_barrier()

        pltpu.sync_copy(
            acc.at[pl.ds(row0, ROWS_PER_TILE)],
            out_hbm.at[cid, pl.ds(row0, ROWS_PER_TILE)],
        )

        @pl.when(sid == NS - 1)
        def _():
            pltpu.sync_copy(
                acc.at[pl.ds(NS * ROWS_PER_TILE, ROWS_TAIL)],
                out_hbm.at[cid, pl.ds(NS * ROWS_PER_TILE, ROWS_TAIL)])

    return k(x, src3, dst3, zeros)


def _combine_tc(x, partials):
    def body(x_ref, p_ref, o_ref):
        o_ref[...] = x_ref[...] + p_ref[0] + p_ref[1]

    rows = 1000
    grid = N_NODES // rows
    return pl.pallas_call(
        body,
        grid=(grid,),
        in_specs=[
            pl.BlockSpec((rows, D_FEAT), lambda i: (i, 0)),
            pl.BlockSpec((NC, rows, D_FEAT), lambda i: (0, i, 0)),
        ],
        out_specs=pl.BlockSpec((rows, D_FEAT), lambda i: (i, 0)),
        out_shape=jax.ShapeDtypeStruct((N_NODES, D_FEAT), jnp.float32),
    )(x, partials)


def kernel(x, edge_index):
    ei = edge_index.astype(jnp.int32)
    pad = EPW_PAD - EPW
    x_ext = jnp.concatenate([x, jnp.zeros((8, D_FEAT), jnp.float32)], axis=0)
    pad_dst = (jnp.arange(NW, dtype=jnp.int32)[:, None] * pad
               + jnp.arange(pad, dtype=jnp.int32)[None, :]) % N_NODES
    srcw = jnp.pad(ei[0].reshape(NW, EPW), ((0, 0), (0, pad)),
                   constant_values=ZERO_ROW)
    dstw = jnp.concatenate([ei[1].reshape(NW, EPW), pad_dst], axis=1)
    src3 = srcw.reshape(NW, NB, B_EDGE)
    dst3 = dstw.reshape(NW, NB, B_EDGE)
    zeros = jnp.zeros((ROWS_PER_TILE, D_FEAT), jnp.float32)
    partials = _segment_sum_sc(x_ext, src3, dst3, zeros)
    return _combine_tc(x, partials)


# speedup vs baseline: 1.8029x; 1.8029x over previous
import functools

import jax
import jax.numpy as jnp
from jax import lax
from jax.experimental import pallas as pl
from jax.experimental.pallas import tpu as pltpu
from jax.experimental.pallas import tpu_sc as plsc

N_NODES = 10000
D_FEAT = 128
N_EDGES = 320000

NC = 2
NS = 16
NW = NC * NS
EPW = N_EDGES // NW
B_EDGE = 80
NB = EPW // B_EDGE
ROWS_PER_TILE = 624
ROWS_TAIL = N_NODES - NS * ROWS_PER_TILE


def _segment_sum_sc(x, src3, dst3, zeros):
    mesh = plsc.VectorSubcoreMesh(core_axis_name="c", subcore_axis_name="s")

    @functools.partial(
        pl.kernel,
        mesh=mesh,
        out_type=jax.ShapeDtypeStruct((NC, N_NODES, D_FEAT), jnp.float32),
        scratch_types=[
            pltpu.VMEM((NB, B_EDGE), jnp.int32),
            pltpu.VMEM((NB, B_EDGE), jnp.int32),
            pltpu.VMEM((B_EDGE, D_FEAT), jnp.float32),
            pltpu.VMEM_SHARED((N_NODES, D_FEAT), jnp.float32),
            pltpu.SemaphoreType.DMA,
        ],
    )
    def k(x_hbm, src_hbm, dst_hbm, zero_hbm, out_hbm,
          src_v, dst_v, rows0, acc, gsem0):
        cid = lax.axis_index("c")
        sid = lax.axis_index("s")
        wid = sid * NC + cid
        row0 = sid * ROWS_PER_TILE

        @pl.when(cid == 0)
        def _():
            pltpu.sync_copy(x_hbm.at[pl.ds(row0, ROWS_PER_TILE)],
                            acc.at[pl.ds(row0, ROWS_PER_TILE)])

            @pl.when(sid == NS - 1)
            def _():
                pltpu.sync_copy(
                    x_hbm.at[pl.ds(NS * ROWS_PER_TILE, ROWS_TAIL)],
                    acc.at[pl.ds(NS * ROWS_PER_TILE, ROWS_TAIL)])

        @pl.when(cid == 1)
        def _():
            pltpu.sync_copy(zero_hbm.at[pl.ds(0, ROWS_PER_TILE)],
                            acc.at[pl.ds(row0, ROWS_PER_TILE)])

            @pl.when(sid == NS - 1)
            def _():
                pltpu.sync_copy(
                    zero_hbm.at[pl.ds(0, ROWS_TAIL)],
                    acc.at[pl.ds(NS * ROWS_PER_TILE, ROWS_TAIL)])

        pltpu.sync_copy(src_hbm.at[wid], src_v)
        pltpu.sync_copy(dst_hbm.at[wid], dst_v)
        plsc.subcore_barrier()

        def body(j, carry):
            pltpu.async_copy(x_hbm.at[src_v.at[j]], rows0, gsem0).wait()
            pltpu.sync_copy(rows0, acc.at[dst_v.at[j]], add=True)
            return carry

        lax.fori_loop(0, NB, body, 0)
        plsc.subcore_barrier()

        pltpu.sync_copy(
            acc.at[pl.ds(row0, ROWS_PER_TILE)],
            out_hbm.at[cid, pl.ds(row0, ROWS_PER_TILE)],
        )

        @pl.when(sid == NS - 1)
        def _():
            pltpu.sync_copy(
                acc.at[pl.ds(NS * ROWS_PER_TILE, ROWS_TAIL)],
                out_hbm.at[cid, pl.ds(NS * ROWS_PER_TILE, ROWS_TAIL)])

    return k(x, src3, dst3, zeros)


def _combine_tc(partials):
    def body(p_ref, o_ref):
        o_ref[...] = p_ref[0] + p_ref[1]

    rows = 1000
    grid = N_NODES // rows
    return pl.pallas_call(
        body,
        grid=(grid,),
        in_specs=[pl.BlockSpec((NC, rows, D_FEAT), lambda i: (0, i, 0))],
        out_specs=pl.BlockSpec((rows, D_FEAT), lambda i: (i, 0)),
        out_shape=jax.ShapeDtypeStruct((N_NODES, D_FEAT), jnp.float32),
    )(partials)


def kernel(x, edge_index):
    ei = edge_index.astype(jnp.int32)
    src3 = ei[0].reshape(NW, NB, B_EDGE)
    dst3 = ei[1].reshape(NW, NB, B_EDGE)
    zeros = jnp.zeros((ROWS_PER_TILE, D_FEAT), jnp.float32)
    partials = _segment_sum_sc(x, src3, dst3, zeros)
    return _combine_tc(partials)


# speedup vs baseline: 2.0066x; 1.1130x over previous
import functools

import jax
import jax.numpy as jnp
from jax import lax
from jax.experimental import pallas as pl
from jax.experimental.pallas import tpu as pltpu
from jax.experimental.pallas import tpu_sc as plsc

N_NODES = 10000
D_FEAT = 128
N_EDGES = 320000

NC = 2
NS = 16
NW = NC * NS
EPW = N_EDGES // NW
B_EDGE = 80
NB = EPW // B_EDGE
ROWS_PER_TILE = 624
ROWS_TAIL = N_NODES - NS * ROWS_PER_TILE


def _segment_sum_sc(x, src3, dst3, zeros):
    mesh = plsc.VectorSubcoreMesh(core_axis_name="c", subcore_axis_name="s")

    @functools.partial(
        pl.kernel,
        mesh=mesh,
        out_type=jax.ShapeDtypeStruct((NC, N_NODES, D_FEAT), jnp.float32),
        scratch_types=[
            pltpu.VMEM((NB, B_EDGE), jnp.int32),
            pltpu.VMEM((64, B_EDGE), jnp.int32),
            pltpu.VMEM((B_EDGE, D_FEAT), jnp.float32),
            pltpu.VMEM((B_EDGE, D_FEAT), jnp.float32),
            pltpu.VMEM_SHARED((N_NODES, D_FEAT), jnp.float32),
            pltpu.SemaphoreType.DMA,
            pltpu.SemaphoreType.DMA,
            pltpu.SemaphoreType.DMA,
        ],
    )
    def k(x_hbm, src_hbm, dst_hbm, zero_hbm, out_hbm,
          src_v, dsth, rows0, rows1, acc, gsem0, gsem1, ssem):
        cid = lax.axis_index("c")
        sid = lax.axis_index("s")
        wid = sid * NC + cid
        row0 = sid * ROWS_PER_TILE

        @pl.when(cid == 0)
        def _():
            pltpu.sync_copy(x_hbm.at[pl.ds(row0, ROWS_PER_TILE)],
                            acc.at[pl.ds(row0, ROWS_PER_TILE)])

            @pl.when(sid == NS - 1)
            def _():
                pltpu.sync_copy(
                    x_hbm.at[pl.ds(NS * ROWS_PER_TILE, ROWS_TAIL)],
                    acc.at[pl.ds(NS * ROWS_PER_TILE, ROWS_TAIL)])

        @pl.when(cid == 1)
        def _():
            pltpu.sync_copy(zero_hbm.at[pl.ds(0, ROWS_PER_TILE)],
                            acc.at[pl.ds(row0, ROWS_PER_TILE)])

            @pl.when(sid == NS - 1)
            def _():
                pltpu.sync_copy(
                    zero_hbm.at[pl.ds(0, ROWS_TAIL)],
                    acc.at[pl.ds(NS * ROWS_PER_TILE, ROWS_TAIL)])

        pltpu.sync_copy(src_hbm.at[wid], src_v)
        plsc.subcore_barrier()

        def make_pair(base):
            def pair(q, carry):
                b = base + 2 * q
                lb = 2 * q
                pltpu.async_copy(x_hbm.at[src_v.at[b]], rows0, gsem0).wait()
                hs = pltpu.async_copy(rows0, acc.at[dsth.at[lb]], ssem,
                                      add=True)
                pltpu.async_copy(x_hbm.at[src_v.at[b + 1]], rows1,
                                 gsem1).wait()
                hs.wait()
                pltpu.sync_copy(rows1, acc.at[dsth.at[lb + 1]], add=True)
                return carry
            return pair

        pltpu.sync_copy(dst_hbm.at[wid, pl.ds(0, 64)], dsth)
        lax.fori_loop(0, 32, make_pair(0), 0)
        pltpu.sync_copy(dst_hbm.at[wid, pl.ds(64, NB - 64)],
                        dsth.at[pl.ds(0, NB - 64)])
        lax.fori_loop(0, 30, make_pair(64), 0)
        pltpu.async_copy(x_hbm.at[src_v.at[NB - 1]], rows0, gsem0).wait()
        pltpu.sync_copy(rows0, acc.at[dsth.at[NB - 1 - 64]], add=True)
        plsc.subcore_barrier()

        pltpu.sync_copy(
            acc.at[pl.ds(row0, ROWS_PER_TILE)],
            out_hbm.at[cid, pl.ds(row0, ROWS_PER_TILE)],
        )

        @pl.when(sid == NS - 1)
        def _():
            pltpu.sync_copy(
                acc.at[pl.ds(NS * ROWS_PER_TILE, ROWS_TAIL)],
                out_hbm.at[cid, pl.ds(NS * ROWS_PER_TILE, ROWS_TAIL)])

    return k(x, src3, dst3, zeros)


def _combine_tc(partials):
    def body(p_ref, o_ref):
        o_ref[...] = p_ref[0] + p_ref[1]

    rows = 1000
    grid = N_NODES // rows
    return pl.pallas_call(
        body,
        grid=(grid,),
        in_specs=[pl.BlockSpec((NC, rows, D_FEAT), lambda i: (0, i, 0))],
        out_specs=pl.BlockSpec((rows, D_FEAT), lambda i: (i, 0)),
        out_shape=jax.ShapeDtypeStruct((N_NODES, D_FEAT), jnp.float32),
    )(partials)


def kernel(x, edge_index):
    ei = edge_index.astype(jnp.int32)
    src3 = ei[0].reshape(NW, NB, B_EDGE)
    dst3 = ei[1].reshape(NW, NB, B_EDGE)
    zeros = jnp.zeros((ROWS_PER_TILE, D_FEAT), jnp.float32)
    partials = _segment_sum_sc(x, src3, dst3, zeros)
    return _combine_tc(partials)


# speedup vs baseline: 2.1723x; 1.0826x over previous
import functools

import jax
import jax.numpy as jnp
from jax import lax
from jax.experimental import pallas as pl
from jax.experimental.pallas import tpu as pltpu
from jax.experimental.pallas import tpu_sc as plsc

N_NODES = 10000
D_FEAT = 128
N_EDGES = 320000

NC = 2
NS = 16
NW = NC * NS
EPW = N_EDGES // NW
B_EDGE = 80
NB = EPW // B_EDGE
ROWS_PER_TILE = 624
ROWS_TAIL = N_NODES - NS * ROWS_PER_TILE


def _segment_sum_sc(x, src3, dst3, zeros):
    mesh = plsc.VectorSubcoreMesh(core_axis_name="c", subcore_axis_name="s")

    @functools.partial(
        pl.kernel,
        mesh=mesh,
        out_type=jax.ShapeDtypeStruct((NC, N_NODES, D_FEAT), jnp.float32),
        scratch_types=[
            pltpu.VMEM((NB, B_EDGE), jnp.int32),
            pltpu.VMEM((64, B_EDGE), jnp.int32),
            pltpu.VMEM((B_EDGE, D_FEAT), jnp.float32),
            pltpu.VMEM((B_EDGE, D_FEAT), jnp.float32),
            pltpu.VMEM_SHARED((N_NODES, D_FEAT), jnp.float32),
            pltpu.SemaphoreType.DMA,
            pltpu.SemaphoreType.DMA,
            pltpu.SemaphoreType.DMA,
        ],
    )
    def k(x_hbm, src_hbm, dst_hbm, zero_hbm, out_hbm,
          src_v, dsth, rows0, rows1, acc, gsem0, gsem1, ssem):
        cid = lax.axis_index("c")
        sid = lax.axis_index("s")
        wid = sid * NC + cid
        row0 = sid * ROWS_PER_TILE

        @pl.when(cid == 0)
        def _():
            pltpu.sync_copy(x_hbm.at[pl.ds(row0, ROWS_PER_TILE)],
                            acc.at[pl.ds(row0, ROWS_PER_TILE)])

            @pl.when(sid == NS - 1)
            def _():
                pltpu.sync_copy(
                    x_hbm.at[pl.ds(NS * ROWS_PER_TILE, ROWS_TAIL)],
                    acc.at[pl.ds(NS * ROWS_PER_TILE, ROWS_TAIL)])

        @pl.when(cid == 1)
        def _():
            pltpu.sync_copy(zero_hbm.at[pl.ds(0, ROWS_PER_TILE)],
                            acc.at[pl.ds(row0, ROWS_PER_TILE)])

            @pl.when(sid == NS - 1)
            def _():
                pltpu.sync_copy(
                    zero_hbm.at[pl.ds(0, ROWS_TAIL)],
                    acc.at[pl.ds(NS * ROWS_PER_TILE, ROWS_TAIL)])

        pltpu.sync_copy(src_hbm.at[wid], src_v)
        plsc.subcore_barrier()

        rows = (rows0, rows1)

        def run_block(base_g, base_l, nu):
            hg = pltpu.async_copy(x_hbm.at[src_v.at[base_g]], rows0, gsem0)
            hs_prev = None
            for k in range(nu):
                hg.wait()
                if hs_prev is not None:
                    hs_prev.wait()
                hs = pltpu.async_copy(rows[k % 2],
                                      acc.at[dsth.at[base_l + k]],
                                      ssem, add=True)
                if k + 1 < nu:
                    hg = pltpu.async_copy(x_hbm.at[src_v.at[base_g + k + 1]],
                                          rows[(k + 1) % 2], gsem0)
                hs_prev = hs
            hs_prev.wait()

        pltpu.sync_copy(dst_hbm.at[wid, pl.ds(0, 64)], dsth)
        lax.fori_loop(
            0, 8, lambda q, c: (run_block(8 * q, 8 * q, 8), c)[1], 0)
        pltpu.sync_copy(dst_hbm.at[wid, pl.ds(64, NB - 64)],
                        dsth.at[pl.ds(0, NB - 64)])
        lax.fori_loop(
            0, 7, lambda q, c: (run_block(64 + 8 * q, 8 * q, 8), c)[1], 0)
        run_block(120, 56, 5)
        plsc.subcore_barrier()

        pltpu.sync_copy(
            acc.at[pl.ds(row0, ROWS_PER_TILE)],
            out_hbm.at[cid, pl.ds(row0, ROWS_PER_TILE)],
        )

        @pl.when(sid == NS - 1)
        def _():
            pltpu.sync_copy(
                acc.at[pl.ds(NS * ROWS_PER_TILE, ROWS_TAIL)],
                out_hbm.at[cid, pl.ds(NS * ROWS_PER_TILE, ROWS_TAIL)])

    return k(x, src3, dst3, zeros)


def _combine_tc(partials):
    def body(p_ref, o_ref):
        o_ref[...] = p_ref[0] + p_ref[1]

    rows = 1000
    grid = N_NODES // rows
    return pl.pallas_call(
        body,
        grid=(grid,),
        in_specs=[pl.BlockSpec((NC, rows, D_FEAT), lambda i: (0, i, 0))],
        out_specs=pl.BlockSpec((rows, D_FEAT), lambda i: (i, 0)),
        out_shape=jax.ShapeDtypeStruct((N_NODES, D_FEAT), jnp.float32),
    )(partials)


def kernel(x, edge_index):
    ei = edge_index.astype(jnp.int32)
    src3 = ei[0].reshape(NW, NB, B_EDGE)
    dst3 = ei[1].reshape(NW, NB, B_EDGE)
    zeros = jnp.zeros((ROWS_PER_TILE, D_FEAT), jnp.float32)
    partials = _segment_sum_sc(x, src3, dst3, zeros)
    return _combine_tc(partials)


# speedup vs baseline: 2.2126x; 1.0186x over previous
import functools

import jax
import jax.numpy as jnp
from jax import lax
from jax.experimental import pallas as pl
from jax.experimental.pallas import tpu as pltpu
from jax.experimental.pallas import tpu_sc as plsc

N_NODES = 10000
D_FEAT = 128
N_EDGES = 320000

NC = 2
NS = 16
NW = NC * NS
EPW = N_EDGES // NW
B_EDGE = 80
NB = EPW // B_EDGE
ROWS_PER_TILE = 624
ROWS_TAIL = N_NODES - NS * ROWS_PER_TILE


def _segment_sum_sc(x, src3, dst3, zeros):
    mesh = plsc.VectorSubcoreMesh(core_axis_name="c", subcore_axis_name="s")

    @functools.partial(
        pl.kernel,
        mesh=mesh,
        out_type=jax.ShapeDtypeStruct((NC, N_NODES, D_FEAT), jnp.float32),
        scratch_types=[
            pltpu.VMEM((NB, B_EDGE), jnp.int32),
            pltpu.VMEM((64, B_EDGE), jnp.int32),
            pltpu.VMEM((B_EDGE, D_FEAT), jnp.float32),
            pltpu.VMEM((B_EDGE, D_FEAT), jnp.float32),
            pltpu.VMEM_SHARED((N_NODES, D_FEAT), jnp.float32),
            pltpu.SemaphoreType.DMA,
            pltpu.SemaphoreType.DMA,
            pltpu.SemaphoreType.DMA,
        ],
    )
    def k(x_hbm, src_hbm, dst_hbm, zero_hbm, out_hbm,
          src_v, dsth, rows0, rows1, acc, gsem0, gsem1, ssem):
        cid = lax.axis_index("c")
        sid = lax.axis_index("s")
        wid = sid * NC + cid
        row0 = sid * ROWS_PER_TILE

        @pl.when(cid == 0)
        def _():
            pltpu.sync_copy(x_hbm.at[pl.ds(row0, ROWS_PER_TILE)],
                            acc.at[pl.ds(row0, ROWS_PER_TILE)])

            @pl.when(sid == NS - 1)
            def _():
                pltpu.sync_copy(
                    x_hbm.at[pl.ds(NS * ROWS_PER_TILE, ROWS_TAIL)],
                    acc.at[pl.ds(NS * ROWS_PER_TILE, ROWS_TAIL)])

        @pl.when(cid == 1)
        def _():
            pltpu.sync_copy(zero_hbm.at[pl.ds(0, ROWS_PER_TILE)],
                            acc.at[pl.ds(row0, ROWS_PER_TILE)])

            @pl.when(sid == NS - 1)
            def _():
                pltpu.sync_copy(
                    zero_hbm.at[pl.ds(0, ROWS_TAIL)],
                    acc.at[pl.ds(NS * ROWS_PER_TILE, ROWS_TAIL)])

        pltpu.sync_copy(src_hbm.at[wid], src_v)
        plsc.subcore_barrier()

        rows = (rows0, rows1)

        def run_block(base_g, base_l, nu):
            hg = pltpu.async_copy(x_hbm.at[src_v.at[base_g]], rows0, gsem0)
            hs_prev = None
            for k in range(nu):
                hg.wait()
                if hs_prev is not None:
                    hs_prev.wait()
                hs = pltpu.async_copy(rows[k % 2],
                                      acc.at[dsth.at[base_l + k]],
                                      ssem, add=True)
                if k + 1 < nu:
                    hg = pltpu.async_copy(x_hbm.at[src_v.at[base_g + k + 1]],
                                          rows[(k + 1) % 2], gsem0)
                hs_prev = hs
            hs_prev.wait()

        pltpu.sync_copy(dst_hbm.at[wid, pl.ds(0, 64)], dsth)
        lax.fori_loop(
            0, 4, lambda q, c: (run_block(16 * q, 16 * q, 16), c)[1], 0)
        pltpu.sync_copy(dst_hbm.at[wid, pl.ds(64, NB - 64)],
                        dsth.at[pl.ds(0, NB - 64)])
        lax.fori_loop(
            0, 3, lambda q, c: (run_block(64 + 16 * q, 16 * q, 16), c)[1], 0)
        run_block(112, 48, 13)
        plsc.subcore_barrier()

        pltpu.sync_copy(
            acc.at[pl.ds(row0, ROWS_PER_TILE)],
            out_hbm.at[cid, pl.ds(row0, ROWS_PER_TILE)],
        )

        @pl.when(sid == NS - 1)
        def _():
            pltpu.sync_copy(
                acc.at[pl.ds(NS * ROWS_PER_TILE, ROWS_TAIL)],
                out_hbm.at[cid, pl.ds(NS * ROWS_PER_TILE, ROWS_TAIL)])

    return k(x, src3, dst3, zeros)


def _combine_tc(partials):
    def body(p_ref, o_ref):
        o_ref[...] = p_ref[0] + p_ref[1]

    rows = 1000
    grid = N_NODES // rows
    return pl.pallas_call(
        body,
        grid=(grid,),
        in_specs=[pl.BlockSpec((NC, rows, D_FEAT), lambda i: (0, i, 0))],
        out_specs=pl.BlockSpec((rows, D_FEAT), lambda i: (i, 0)),
        out_shape=jax.ShapeDtypeStruct((N_NODES, D_FEAT), jnp.float32),
    )(partials)


def kernel(x, edge_index):
    ei = edge_index.astype(jnp.int32)
    src3 = ei[0].reshape(NW, NB, B_EDGE)
    dst3 = ei[1].reshape(NW, NB, B_EDGE)
    zeros = jnp.zeros((ROWS_PER_TILE, D_FEAT), jnp.float32)
    partials = _segment_sum_sc(x, src3, dst3, zeros)
    return _combine_tc(partials)


# speedup vs baseline: 2.2253x; 1.0057x over previous
import functools

import jax
import jax.numpy as jnp
from jax import lax
from jax.experimental import pallas as pl
from jax.experimental.pallas import tpu as pltpu
from jax.experimental.pallas import tpu_sc as plsc

N_NODES = 10000
D_FEAT = 128
N_EDGES = 320000

NC = 2
NS = 16
NW = NC * NS
EPW = N_EDGES // NW
B_EDGE = 80
NB = EPW // B_EDGE
ROWS_PER_TILE = 624
ROWS_TAIL = N_NODES - NS * ROWS_PER_TILE


def _segment_sum_sc(x, src3, dst3, zeros):
    mesh = plsc.VectorSubcoreMesh(core_axis_name="c", subcore_axis_name="s")

    @functools.partial(
        pl.kernel,
        mesh=mesh,
        out_type=jax.ShapeDtypeStruct((NC, N_NODES, D_FEAT), jnp.float32),
        scratch_types=[
            pltpu.VMEM((NB, B_EDGE), jnp.int32),
            pltpu.VMEM((64, B_EDGE), jnp.int32),
            pltpu.VMEM((B_EDGE, D_FEAT), jnp.float32),
            pltpu.VMEM((B_EDGE, D_FEAT), jnp.float32),
            pltpu.VMEM_SHARED((N_NODES, D_FEAT), jnp.float32),
            pltpu.SemaphoreType.DMA,
            pltpu.SemaphoreType.DMA,
            pltpu.SemaphoreType.DMA,
        ],
    )
    def k(x_hbm, src_hbm, dst_hbm, zero_hbm, out_hbm,
          src_v, dsth, rows0, rows1, acc, gsem0, gsem1, ssem):
        cid = lax.axis_index("c")
        sid = lax.axis_index("s")
        wid = sid * NC + cid
        row0 = sid * ROWS_PER_TILE

        @pl.when(cid == 0)
        def _():
            pltpu.sync_copy(x_hbm.at[pl.ds(row0, ROWS_PER_TILE)],
                            acc.at[pl.ds(row0, ROWS_PER_TILE)])

            @pl.when(sid == NS - 1)
            def _():
                pltpu.sync_copy(
                    x_hbm.at[pl.ds(NS * ROWS_PER_TILE, ROWS_TAIL)],
                    acc.at[pl.ds(NS * ROWS_PER_TILE, ROWS_TAIL)])

        @pl.when(cid == 1)
        def _():
            pltpu.sync_copy(zero_hbm.at[pl.ds(0, ROWS_PER_TILE)],
                            acc.at[pl.ds(row0, ROWS_PER_TILE)])

            @pl.when(sid == NS - 1)
            def _():
                pltpu.sync_copy(
                    zero_hbm.at[pl.ds(0, ROWS_TAIL)],
                    acc.at[pl.ds(NS * ROWS_PER_TILE, ROWS_TAIL)])

        pltpu.sync_copy(src_hbm.at[wid], src_v)
        plsc.subcore_barrier()

        rows = (rows0, rows1)

        def run_block(base_g, base_l, nu):
            hg = pltpu.async_copy(x_hbm.at[src_v.at[base_g]], rows0, gsem0)
            hs_prev = None
            for k in range(nu):
                hg.wait()
                if hs_prev is not None:
                    hs_prev.wait()
                hs = pltpu.async_copy(rows[k % 2],
                                      acc.at[dsth.at[base_l + k]],
                                      ssem, add=True)
                if k + 1 < nu:
                    hg = pltpu.async_copy(x_hbm.at[src_v.at[base_g + k + 1]],
                                          rows[(k + 1) % 2], gsem0)
                hs_prev = hs
            hs_prev.wait()

        pltpu.sync_copy(dst_hbm.at[wid, pl.ds(0, 64)], dsth)
        lax.fori_loop(
            0, 2, lambda q, c: (run_block(32 * q, 32 * q, 32), c)[1], 0)
        pltpu.sync_copy(dst_hbm.at[wid, pl.ds(64, NB - 64)],
                        dsth.at[pl.ds(0, NB - 64)])
        run_block(64, 0, 32)
        run_block(96, 32, 29)
        plsc.subcore_barrier()

        pltpu.sync_copy(
            acc.at[pl.ds(row0, ROWS_PER_TILE)],
            out_hbm.at[cid, pl.ds(row0, ROWS_PER_TILE)],
        )

        @pl.when(sid == NS - 1)
        def _():
            pltpu.sync_copy(
                acc.at[pl.ds(NS * ROWS_PER_TILE, ROWS_TAIL)],
                out_hbm.at[cid, pl.ds(NS * ROWS_PER_TILE, ROWS_TAIL)])

    return k(x, src3, dst3, zeros)


def _combine_tc(partials):
    def body(p_ref, o_ref):
        o_ref[...] = p_ref[0] + p_ref[1]

    rows = 1000
    grid = N_NODES // rows
    return pl.pallas_call(
        body,
        grid=(grid,),
        in_specs=[pl.BlockSpec((NC, rows, D_FEAT), lambda i: (0, i, 0))],
        out_specs=pl.BlockSpec((rows, D_FEAT), lambda i: (i, 0)),
        out_shape=jax.ShapeDtypeStruct((N_NODES, D_FEAT), jnp.float32),
    )(partials)


def kernel(x, edge_index):
    ei = edge_index.astype(jnp.int32)
    src3 = ei[0].reshape(NW, NB, B_EDGE)
    dst3 = ei[1].reshape(NW, NB, B_EDGE)
    zeros = jnp.zeros((ROWS_PER_TILE, D_FEAT), jnp.float32)
    partials = _segment_sum_sc(x, src3, dst3, zeros)
    return _combine_tc(partials)


# speedup vs baseline: 2.2547x; 1.0132x over previous
import functools

import jax
import jax.numpy as jnp
from jax import lax
from jax.experimental import pallas as pl
from jax.experimental.pallas import tpu as pltpu
from jax.experimental.pallas import tpu_sc as plsc

N_NODES = 10000
D_FEAT = 128
N_EDGES = 320000

NC = 2
NS = 16
NW = NC * NS
EPW = N_EDGES // NW
B_EDGE = 80
NB = EPW // B_EDGE
ROWS_PER_TILE = 624
ROWS_TAIL = N_NODES - NS * ROWS_PER_TILE


def _segment_sum_sc(x, src3, dst3, zeros):
    mesh = plsc.VectorSubcoreMesh(core_axis_name="c", subcore_axis_name="s")

    @functools.partial(
        pl.kernel,
        mesh=mesh,
        out_type=jax.ShapeDtypeStruct((NC, N_NODES, D_FEAT), jnp.float32),
        scratch_types=[
            pltpu.VMEM((NB, B_EDGE), jnp.int32),
            pltpu.VMEM((64, B_EDGE), jnp.int32),
            pltpu.VMEM((B_EDGE, D_FEAT), jnp.float32),
            pltpu.VMEM((B_EDGE, D_FEAT), jnp.float32),
            pltpu.VMEM_SHARED((N_NODES, D_FEAT), jnp.float32),
            pltpu.SemaphoreType.DMA,
            pltpu.SemaphoreType.DMA,
            pltpu.SemaphoreType.DMA,
        ],
    )
    def k(x_hbm, src_hbm, dst_hbm, zero_hbm, out_hbm,
          src_v, dsth, rows0, rows1, acc, gsem0, gsem1, ssem):
        cid = lax.axis_index("c")
        sid = lax.axis_index("s")
        wid = sid * NC + cid
        row0 = sid * ROWS_PER_TILE

        h_src = pltpu.async_copy(src_hbm.at[wid], src_v, gsem1)
        h_dst0 = pltpu.async_copy(dst_hbm.at[wid, pl.ds(0, 64)], dsth, ssem)

        @pl.when(cid == 0)
        def _():
            pltpu.sync_copy(x_hbm.at[pl.ds(row0, ROWS_PER_TILE)],
                            acc.at[pl.ds(row0, ROWS_PER_TILE)])

            @pl.when(sid == NS - 1)
            def _():
                pltpu.sync_copy(
                    x_hbm.at[pl.ds(NS * ROWS_PER_TILE, ROWS_TAIL)],
                    acc.at[pl.ds(NS * ROWS_PER_TILE, ROWS_TAIL)])

        @pl.when(cid == 1)
        def _():
            pltpu.sync_copy(zero_hbm.at[pl.ds(0, ROWS_PER_TILE)],
                            acc.at[pl.ds(row0, ROWS_PER_TILE)])

            @pl.when(sid == NS - 1)
            def _():
                pltpu.sync_copy(
                    zero_hbm.at[pl.ds(0, ROWS_TAIL)],
                    acc.at[pl.ds(NS * ROWS_PER_TILE, ROWS_TAIL)])

        h_src.wait()
        h_dst0.wait()
        plsc.subcore_barrier()

        rows = (rows0, rows1)

        def run_block(base_g, base_l, nu):
            hg = pltpu.async_copy(x_hbm.at[src_v.at[base_g]], rows0, gsem0)
            hs_prev = None
            for k in range(nu):
                hg.wait()
                if hs_prev is not None:
                    hs_prev.wait()
                hs = pltpu.async_copy(rows[k % 2],
                                      acc.at[dsth.at[base_l + k]],
                                      ssem, add=True)
                if k + 1 < nu:
                    hg = pltpu.async_copy(x_hbm.at[src_v.at[base_g + k + 1]],
                                          rows[(k + 1) % 2], gsem0)
                hs_prev = hs
            hs_prev.wait()

        lax.fori_loop(
            0, 2, lambda q, c: (run_block(32 * q, 32 * q, 32), c)[1], 0)
        pltpu.sync_copy(dst_hbm.at[wid, pl.ds(64, NB - 64)],
                        dsth.at[pl.ds(0, NB - 64)])
        run_block(64, 0, 32)
        run_block(96, 32, 29)
        plsc.subcore_barrier()

        pltpu.sync_copy(
            acc.at[pl.ds(row0, ROWS_PER_TILE)],
            out_hbm.at[cid, pl.ds(row0, ROWS_PER_TILE)],
        )

        @pl.when(sid == NS - 1)
        def _():
            pltpu.sync_copy(
                acc.at[pl.ds(NS * ROWS_PER_TILE, ROWS_TAIL)],
                out_hbm.at[cid, pl.ds(NS * ROWS_PER_TILE, ROWS_TAIL)])

    return k(x, src3, dst3, zeros)


def _combine_tc(partials):
    def body(p_ref, o_ref):
        o_ref[...] = p_ref[0] + p_ref[1]

    rows = 1000
    grid = N_NODES // rows
    return pl.pallas_call(
        body,
        grid=(grid,),
        in_specs=[pl.BlockSpec((NC, rows, D_FEAT), lambda i: (0, i, 0))],
        out_specs=pl.BlockSpec((rows, D_FEAT), lambda i: (i, 0)),
        out_shape=jax.ShapeDtypeStruct((N_NODES, D_FEAT), jnp.float32),
    )(partials)


def kernel(x, edge_index):
    ei = edge_index.astype(jnp.int32)
    src3 = ei[0].reshape(NW, NB, B_EDGE)
    dst3 = ei[1].reshape(NW, NB, B_EDGE)
    zeros = jnp.zeros((ROWS_PER_TILE, D_FEAT), jnp.float32)
    partials = _segment_sum_sc(x, src3, dst3, zeros)
    return _combine_tc(partials)
